# baseline (speedup 1.0000x reference)
import functools

import jax
import jax.numpy as jnp
from jax import lax
from jax.experimental import pallas as pl
from jax.experimental.pallas import tpu as pltpu
from jax.experimental.pallas import tpu_sc as plsc

_N = 16777216
_NW = 32
_CHUNK = _N // _NW
_ZBUF = 65536
_NZ = _CHUNK // _ZBUF

_mesh = plsc.VectorSubcoreMesh(core_axis_name="c", subcore_axis_name="s")


@functools.partial(
    pl.kernel,
    mesh=_mesh,
    out_type=jax.ShapeDtypeStruct((2 * _N,), jnp.float32),
    scratch_types=[pltpu.VMEM((_ZBUF,), jnp.float32)],
)
def _sc_kernel(psi_hbm, out_hbm, zbuf):
    wid = lax.axis_index("s") * 2 + lax.axis_index("c")
    base = wid * _CHUNK

    def zstep(i, carry):
        zbuf[pl.ds(i * 16, 16)] = jnp.zeros((16,), jnp.float32)
        return carry

    lax.fori_loop(0, _ZBUF // 16, zstep, 0)

    pltpu.sync_copy(psi_hbm.at[pl.ds(base, _CHUNK)],
                    out_hbm.at[pl.ds(base, _CHUNK)])

    def zdma(j, carry):
        pltpu.sync_copy(zbuf,
                        out_hbm.at[pl.ds(_N + base + j * _ZBUF, _ZBUF)])
        return carry

    lax.fori_loop(0, _NZ, zdma, 0)


def kernel(psi):
    return _sc_kernel(psi)


# speedup vs baseline: 1.0187x; 1.0187x over previous
import functools

import jax
import jax.numpy as jnp
from jax import lax
from jax.experimental import pallas as pl
from jax.experimental.pallas import tpu as pltpu
from jax.experimental.pallas import tpu_sc as plsc

_N = 16777216
_NW = 32
_CHUNK = _N // _NW
_ZBUF = 65536
_NZ = _CHUNK // _ZBUF

_mesh = plsc.VectorSubcoreMesh(core_axis_name="c", subcore_axis_name="s")


@functools.partial(
    pl.kernel,
    mesh=_mesh,
    out_type=jax.ShapeDtypeStruct((2 * _N,), jnp.float32),
    scratch_types=[
        pltpu.VMEM((_ZBUF,), jnp.float32),
        pltpu.SemaphoreType.DMA,
        pltpu.SemaphoreType.DMA,
    ],
)
def _sc_kernel(psi_hbm, out_hbm, zbuf, csem, zsem):
    wid = lax.axis_index("s") * 2 + lax.axis_index("c")
    base = wid * _CHUNK

    copy = pltpu.make_async_copy(psi_hbm.at[pl.ds(base, _CHUNK)],
                                 out_hbm.at[pl.ds(base, _CHUNK)], csem)
    copy.start()

    def zstep(i, carry):
        for k in range(16):
            zbuf[pl.ds(i * 256 + k * 16, 16)] = jnp.zeros((16,), jnp.float32)
        return carry

    lax.fori_loop(0, _ZBUF // 256, zstep, 0)

    zcopies = []
    for j in range(_NZ):
        d = pltpu.make_async_copy(
            zbuf, out_hbm.at[pl.ds(_N + base + j * _ZBUF, _ZBUF)], zsem)
        d.start()
        zcopies.append(d)

    copy.wait()
    for d in zcopies:
        d.wait()


def kernel(psi):
    return _sc_kernel(psi)


# speedup vs baseline: 22.7305x; 22.3125x over previous
import functools

import jax
import jax.numpy as jnp
from jax import lax
from jax.experimental import pallas as pl
from jax.experimental.pallas import tpu as pltpu
from jax.experimental.pallas import tpu_sc as plsc

_N = 16777216
_NW = 32
_CHUNK = _N // _NW
_PIECE = 16384
_NP = _CHUNK // _PIECE
_NBUF = 4

_mesh = plsc.VectorSubcoreMesh(core_axis_name="c", subcore_axis_name="s")


@functools.partial(
    pl.kernel,
    mesh=_mesh,
    out_type=jax.ShapeDtypeStruct((2 * _N,), jnp.float32),
    scratch_types=(
        [pltpu.VMEM((_PIECE,), jnp.float32)]
        + [pltpu.VMEM((_PIECE,), jnp.float32)] * _NBUF
        + [pltpu.SemaphoreType.DMA]
        + [pltpu.SemaphoreType.DMA] * _NBUF
        + [pltpu.SemaphoreType.DMA]
    ),
)
def _sc_kernel(psi_hbm, out_hbm, zbuf, b0, b1, b2, b3,
               gsem, s0, s1, s2, s3, zsem):
    bufs = [b0, b1, b2, b3]
    ssems = [s0, s1, s2, s3]
    wid = lax.axis_index("s") * 2 + lax.axis_index("c")
    base = wid * _CHUNK

    def gather(i, b):
        return pltpu.make_async_copy(
            psi_hbm.at[pl.ds(base + i * _PIECE, _PIECE)], bufs[b], gsem)

    def scatter(i, b):
        return pltpu.make_async_copy(
            bufs[b], out_hbm.at[pl.ds(base + i * _PIECE, _PIECE)], ssems[b])

    gathers = [gather(i, i % _NBUF) for i in range(_NBUF)]
    for g in gathers:
        g.start()

    def zstep(i, carry):
        for k in range(16):
            zbuf[pl.ds(i * 256 + k * 16, 16)] = jnp.zeros((16,), jnp.float32)
        return carry

    lax.fori_loop(0, _PIECE // 256, zstep, 0)

    zeros = [
        pltpu.make_async_copy(
            zbuf, out_hbm.at[pl.ds(_N + base + j * _PIECE, _PIECE)], zsem)
        for j in range(_NP)
    ]
    for z in zeros:
        z.start()

    scatters = []
    for i in range(_NP):
        b = i % _NBUF
        gathers[i].wait()
        sc = scatter(i, b)
        sc.start()
        scatters.append(sc)
        ni = i + _NBUF
        if ni < _NP:
            sc.wait()
            g = gather(ni, b)
            g.start()
            gathers.append(g)

    for sc in scatters[_NP - _NBUF:]:
        sc.wait()
    for z in zeros:
        z.wait()


def kernel(psi):
    return _sc_kernel(psi)


# speedup vs baseline: 22.9446x; 1.0094x over previous
import functools

import jax
import jax.numpy as jnp
from jax import lax
from jax.experimental import pallas as pl
from jax.experimental.pallas import tpu as pltpu
from jax.experimental.pallas import tpu_sc as plsc

_N = 16777216
_NW = 32
_CHUNK = _N // _NW
_PIECE = 32768
_NP = _CHUNK // _PIECE
_NBUF = 2

_mesh = plsc.VectorSubcoreMesh(core_axis_name="c", subcore_axis_name="s")


@functools.partial(
    pl.kernel,
    mesh=_mesh,
    out_type=jax.ShapeDtypeStruct((2 * _N,), jnp.float32),
    scratch_types=(
        [pltpu.VMEM((_PIECE,), jnp.float32)]
        + [pltpu.VMEM((_PIECE,), jnp.float32)] * _NBUF
        + [pltpu.SemaphoreType.DMA]
        + [pltpu.SemaphoreType.DMA] * _NBUF
        + [pltpu.SemaphoreType.DMA]
    ),
)
def _sc_kernel(psi_hbm, out_hbm, zbuf, *scratch):
    bufs = list(scratch[:_NBUF])
    gsem = scratch[_NBUF]
    ssems = list(scratch[_NBUF + 1:_NBUF + 1 + _NBUF])
    zsem = scratch[_NBUF + 1 + _NBUF]
    wid = lax.axis_index("s") * 2 + lax.axis_index("c")
    base = wid * _CHUNK

    def gather(i, b):
        return pltpu.make_async_copy(
            psi_hbm.at[pl.ds(base + i * _PIECE, _PIECE)], bufs[b], gsem)

    def scatter(i, b):
        return pltpu.make_async_copy(
            bufs[b], out_hbm.at[pl.ds(base + i * _PIECE, _PIECE)], ssems[b])

    gathers = [gather(i, i % _NBUF) for i in range(_NBUF)]
    for g in gathers:
        g.start()

    def zstep(i, carry):
        for k in range(16):
            zbuf[pl.ds(i * 256 + k * 16, 16)] = jnp.zeros((16,), jnp.float32)
        return carry

    lax.fori_loop(0, _PIECE // 256, zstep, 0)

    zeros = [
        pltpu.make_async_copy(
            zbuf, out_hbm.at[pl.ds(_N + base + j * _PIECE, _PIECE)], zsem)
        for j in range(_NP)
    ]
    for z in zeros:
        z.start()

    scatters = []
    for i in range(_NP):
        b = i % _NBUF
        gathers[i].wait()
        sc = scatter(i, b)
        sc.start()
        scatters.append(sc)
        ni = i + _NBUF
        if ni < _NP:
            sc.wait()
            g = gather(ni, b)
            g.start()
            gathers.append(g)

    for sc in scatters[_NP - _NBUF:]:
        sc.wait()
    for z in zeros:
        z.wait()


def kernel(psi):
    return _sc_kernel(psi)
